# baseline (speedup 1.0000x reference)
import jax
import jax.numpy as jnp
from jax.experimental import pallas as pl
from jax.experimental.pallas import tpu as pltpu

_BM = 400


def _support_body(x_ref, w_ref, out_ref):
    out_ref[...] = jnp.dot(x_ref[...], w_ref[...], preferred_element_type=jnp.float32)


def _spmm_body(adj_ref, s_ref, b_ref, out_ref):
    out_ref[...] = (
        jnp.dot(adj_ref[...], s_ref[...], preferred_element_type=jnp.float32)
        + b_ref[...]
    )


def kernel(input, adj, weight, bias):
    n, k = input.shape
    m = adj.shape[0]
    f = weight.shape[1]
    bias2 = bias.reshape(1, f)

    support = pl.pallas_call(
        _support_body,
        out_shape=jax.ShapeDtypeStruct((n, f), jnp.float32),
    )(input, weight)

    return pl.pallas_call(
        _spmm_body,
        grid=(m // _BM,),
        in_specs=[
            pl.BlockSpec((_BM, n), lambda i: (i, 0)),
            pl.BlockSpec((n, f), lambda i: (0, 0)),
            pl.BlockSpec((1, f), lambda i: (0, 0)),
        ],
        out_specs=pl.BlockSpec((_BM, f), lambda i: (i, 0)),
        out_shape=jax.ShapeDtypeStruct((m, f), jnp.float32),
    )(adj, support, bias2)


# speedup vs baseline: 1.0310x; 1.0310x over previous
import jax
import jax.numpy as jnp
from jax.experimental import pallas as pl
from jax.experimental.pallas import tpu as pltpu

_BM = 400


def _gcn_body(x_ref, adj_ref, w_ref, b_ref, out_ref, support_ref):
    @pl.when(pl.program_id(0) == 0)
    def _():
        support_ref[...] = jnp.dot(
            x_ref[...], w_ref[...], preferred_element_type=jnp.float32
        )

    out_ref[...] = (
        jnp.dot(adj_ref[...], support_ref[...], preferred_element_type=jnp.float32)
        + b_ref[...]
    )


def kernel(input, adj, weight, bias):
    n, k = input.shape
    m = adj.shape[0]
    f = weight.shape[1]

    return pl.pallas_call(
        _gcn_body,
        grid=(m // _BM,),
        in_specs=[
            pl.BlockSpec((n, k), lambda i: (0, 0)),
            pl.BlockSpec((_BM, n), lambda i: (i, 0)),
            pl.BlockSpec((k, f), lambda i: (0, 0)),
            pl.BlockSpec((f,), lambda i: (0,)),
        ],
        out_specs=pl.BlockSpec((_BM, f), lambda i: (i, 0)),
        out_shape=jax.ShapeDtypeStruct((m, f), jnp.float32),
        scratch_shapes=[pltpu.VMEM((n, f), jnp.float32)],
    )(input, adj, weight, bias)


# speedup vs baseline: 1.0862x; 1.0535x over previous
import jax
import jax.numpy as jnp
from jax.experimental import pallas as pl
from jax.experimental.pallas import tpu as pltpu

_BM = 400


def _gcn_body(x_ref, adj_ref, wt_ref, b_ref, out_ref, support_ref, acc_ref):
    i = pl.program_id(0)

    @pl.when(i == 0)
    def _():
        support_ref[...] = jax.lax.dot_general(
            x_ref[...],
            wt_ref[...],
            (((1,), (1,)), ((), ())),
            preferred_element_type=jnp.float32,
        )

    blk = (
        jax.lax.dot_general(
            adj_ref[...],
            support_ref[...],
            (((1,), (0,)), ((), ())),
            preferred_element_type=jnp.float32,
        )
        + b_ref[...]
    )
    acc_ref[pl.ds(i * _BM, _BM), :] = blk

    @pl.when(i == pl.num_programs(0) - 1)
    def _():
        out_ref[...] = acc_ref[...].T


def kernel(input, adj, weight, bias):
    n, k = input.shape
    m = adj.shape[0]
    f = weight.shape[1]

    out_t = pl.pallas_call(
        _gcn_body,
        grid=(m // _BM,),
        in_specs=[
            pl.BlockSpec((n, k), lambda i: (0, 0)),
            pl.BlockSpec((_BM, n), lambda i: (i, 0)),
            pl.BlockSpec((f, k), lambda i: (0, 0)),
            pl.BlockSpec((1, f), lambda i: (0, 0)),
        ],
        out_specs=pl.BlockSpec((f, m), lambda i: (0, 0)),
        out_shape=jax.ShapeDtypeStruct((f, m), jnp.float32),
        scratch_shapes=[
            pltpu.VMEM((n, f), jnp.float32),
            pltpu.VMEM((m, f), jnp.float32),
        ],
    )(input, adj, weight.T, bias.reshape(1, f))
    return out_t.T
